# baseline (speedup 1.0000x reference)
import functools
import math

import jax
import jax.numpy as jnp
from jax import lax
from jax.experimental import pallas as pl
from jax.experimental.pallas import tpu as pltpu
from jax.experimental.pallas import tpu_sc as plsc

_SCALE = 8.0
_LANES = 16
_NBUF = 3


@functools.cache
def _build(B0, S, V, D):
    NC, NS = 2, 16
    NW = NC * NS
    assert B0 % NW == 0
    bt = B0 // NW
    NB = 8
    assert bt % NB == 0
    n_chunks = bt // NB
    vecs = (NB * S * D) // _LANES
    kpr = D // _LANES
    main_end = 1 + ((n_chunks - 3 - 1) // _NBUF) * _NBUF
    assert main_end >= 1 and main_end + 2 <= n_chunks

    mesh = plsc.VectorSubcoreMesh(core_axis_name="c", subcore_axis_name="s")

    @functools.partial(
        pl.kernel,
        mesh=mesh,
        compiler_params=pltpu.CompilerParams(use_tc_tiling_on_sc=False),
        out_type=jax.ShapeDtypeStruct((B0, S, D), jnp.float32),
        scratch_types=[
            [pltpu.VMEM((NB, S), jnp.int32) for _ in range(_NBUF)],
            [pltpu.VMEM((NB, S, D), jnp.float32) for _ in range(_NBUF)],
            [pltpu.SemaphoreType.DMA for _ in range(_NBUF)],
            [pltpu.SemaphoreType.DMA for _ in range(_NBUF)],
        ],
    )
    def emb(x_hbm, lut_hbm, out_hbm, idx_v, rows_v, gsem, ssem):
        wid = lax.axis_index("s") * NC + lax.axis_index("c")
        base = wid * bt

        def load_and_gather(c, b):
            pltpu.sync_copy(x_hbm.at[pl.ds(base + c * NB, NB), :], idx_v[b])
            for j in range(NB):
                pltpu.async_copy(
                    lut_hbm.at[idx_v[b].at[j]], rows_v[b].at[j], gsem[b]
                )

        def drain_gather(b):
            pltpu.make_async_copy(
                out_hbm.at[pl.ds(0, NB)], rows_v[b], gsem[b]
            ).wait()

        def start_store(c, b):
            pltpu.async_copy(
                rows_v[b], out_hbm.at[pl.ds(base + c * NB, NB)], ssem[b]
            )

        def drain_store(b):
            pltpu.make_async_copy(
                rows_v[b], out_hbm.at[pl.ds(0, NB)], ssem[b]
            ).wait()

        def scale(b):
            for r in range(NB):
                @plsc.parallel_loop(0, S * kpr, unroll=8)
                def _(i, r=r):
                    s = i // kpr
                    k = i % kpr
                    sl = pl.ds(k * _LANES, _LANES)
                    rows_v[b][r, s, sl] = rows_v[b][r, s, sl] * _SCALE

        load_and_gather(0, 0)
        load_and_gather(1, 1)
        drain_gather(0)
        load_and_gather(2, 2)
        scale(0)
        start_store(0, 0)

        @pl.loop(1, main_end, step=_NBUF)
        def _(i):
            for b_off in range(_NBUF):
                c = i + b_off
                b = (1 + b_off) % _NBUF
                nb = (b + 2) % _NBUF
                drain_gather(b)
                drain_store(nb)
                load_and_gather(c + 2, nb)
                scale(b)
                start_store(c, b)

        for c in range(main_end, n_chunks):
            b = c % _NBUF
            nb = (b + 2) % _NBUF
            drain_gather(b)
            if c + 2 < n_chunks:
                drain_store(nb)
                load_and_gather(c + 2, nb)
            scale(b)
            start_store(c, b)
        for c in range(n_chunks - _NBUF, n_chunks):
            drain_store(c % _NBUF)

    return emb


def kernel(x, lut):
    B0, S = x.shape
    V, D = lut.shape
    return _build(B0, S, V, D)(x.astype(jnp.int32), lut)
